# baseline (speedup 1.0000x reference)
import functools

import jax
import jax.numpy as jnp
from jax import lax
from jax.experimental import pallas as pl
from jax.experimental.pallas import tpu as pltpu
from jax.experimental.pallas import tpu_sc as plsc

BATCH = 4
SENT = 8192
FEAT = 768
NUM_WORKERS = 32
ROWS_PER_WORKER = SENT // NUM_WORKERS
CHUNK = 16
NUM_CHUNKS = ROWS_PER_WORKER // CHUNK
LANES = 16
SLICES = FEAT // LANES


def _pe_body(in_hbm, pos_hbm, out_hbm, buf, sem):
    iob = [buf.at[i] for i in range(8)]
    pob = [buf.at[8 + i] for i in range(2)]
    sin = [sem.at[i] for i in range(4)]
    sou = [sem.at[4 + i] for i in range(4)]
    spo = [sem.at[8 + i] for i in range(2)]

    wid = lax.axis_index("s") * 2 + lax.axis_index("c")
    base = wid * ROWS_PER_WORKER

    def in_copy(c, b, par):
        row0 = base + c * CHUNK
        return pltpu.make_async_copy(
            in_hbm.at[b, pl.ds(row0, CHUNK)], iob[b * 2 + par], sin[b])

    def out_copy(c, b, par):
        row0 = base + c * CHUNK
        return pltpu.make_async_copy(
            iob[b * 2 + par], out_hbm.at[b, pl.ds(row0, CHUNK)], sou[b])

    def pos_copy(c, par):
        row0 = base + c * CHUNK
        return pltpu.make_async_copy(
            pos_hbm.at[pl.ds(row0, CHUNK)], pob[par], spo[par])

    for b in range(BATCH):
        in_copy(0, b, 0).start()
    pos_copy(0, 0).start()

    def pair_body(cc, carry):
        for P in range(2):
            c = cc * 2 + P

            pos_copy(c, P).wait()
            if P == 1:
                @pl.when(cc < NUM_CHUNKS // 2 - 1)
                def _():
                    pos_copy(c + 1, 1 - P).start()
            else:
                pos_copy(c + 1, 1 - P).start()

            for b in range(BATCH):
                in_copy(c, b, P).wait()
            for b in range(BATCH):
                if P == 0:
                    @pl.when(cc > 0)
                    def _():
                        out_copy(c - 1, b, 1 - P).wait()

                    in_copy(c + 1, b, 1 - P).start()
                else:
                    out_copy(c - 1, b, 1 - P).wait()

                    @pl.when(cc < NUM_CHUNKS // 2 - 1)
                    def _():
                        in_copy(c + 1, b, 1 - P).start()

            def row_body(r, rc):
                for j in range(SLICES):
                    sl = pl.ds(j * LANES, LANES)
                    p = pob[P][r, sl]
                    for b in range(BATCH):
                        iob[b * 2 + P][r, sl] = iob[b * 2 + P][r, sl] + p
                return rc

            H = CHUNK // 2
            lax.fori_loop(0, H, row_body, 0)
            for b in range(BATCH):
                row0 = base + c * CHUNK
                pltpu.make_async_copy(
                    iob[b * 2 + P].at[pl.ds(0, H)],
                    out_hbm.at[b, pl.ds(row0, H)], sou[b]).start()
            lax.fori_loop(H, CHUNK, row_body, 0)
            for b in range(BATCH):
                row0 = base + c * CHUNK
                pltpu.make_async_copy(
                    iob[b * 2 + P].at[pl.ds(H, H)],
                    out_hbm.at[b, pl.ds(row0 + H, H)], sou[b]).start()
        return carry

    lax.fori_loop(0, NUM_CHUNKS // 2, pair_body, 0)

    for b in range(BATCH):
        out_copy(NUM_CHUNKS - 1, b, 1).wait()


@functools.partial(
    pl.kernel,
    mesh=plsc.VectorSubcoreMesh(core_axis_name="c", subcore_axis_name="s"),
    out_type=jax.ShapeDtypeStruct((BATCH, SENT, FEAT), jnp.float32),
    scratch_types=[
        pltpu.VMEM((10, CHUNK, FEAT), jnp.float32),
        pltpu.SemaphoreType.DMA((10,)),
    ],
)
def _pe(*refs):
    _pe_body(*refs)


def kernel(inputs, pos_weight):
    return _pe(inputs, pos_weight)


# speedup vs baseline: 1.0144x; 1.0144x over previous
import functools

import jax
import jax.numpy as jnp
from jax import lax
from jax.experimental import pallas as pl
from jax.experimental.pallas import tpu as pltpu
from jax.experimental.pallas import tpu_sc as plsc

BATCH = 4
SENT = 8192
FEAT = 768
NUM_WORKERS = 32
ROWS_PER_WORKER = SENT // NUM_WORKERS
CHUNK = 16
NUM_CHUNKS = ROWS_PER_WORKER // CHUNK
LANES = 16
SLICES = FEAT // LANES


def _pe_body(in_hbm, pos_hbm, out_hbm, buf, sem):
    iob = [buf.at[i] for i in range(8)]
    pob = [buf.at[8 + i] for i in range(2)]
    sin = [sem.at[i] for i in range(4)]
    sou = [sem.at[4 + i] for i in range(4)]
    spo = [sem.at[8 + i] for i in range(2)]

    wid = lax.axis_index("s") * 2 + lax.axis_index("c")
    base = wid * ROWS_PER_WORKER

    def in_copy(c, b, par):
        row0 = base + c * CHUNK
        return pltpu.make_async_copy(
            in_hbm.at[b, pl.ds(row0, CHUNK)], iob[b * 2 + par], sin[b])

    def out_copy(c, b, par):
        row0 = base + c * CHUNK
        return pltpu.make_async_copy(
            iob[b * 2 + par], out_hbm.at[b, pl.ds(row0, CHUNK)], sou[b])

    def pos_copy(c, par):
        row0 = base + c * CHUNK
        return pltpu.make_async_copy(
            pos_hbm.at[pl.ds(row0, CHUNK)], pob[par], spo[par])

    for b in range(BATCH):
        in_copy(0, b, 0).start()
    pos_copy(0, 0).start()

    def pair_body(cc, carry):
        for P in range(2):
            c = cc * 2 + P

            pos_copy(c, P).wait()
            if P == 1:
                @pl.when(cc < NUM_CHUNKS // 2 - 1)
                def _():
                    pos_copy(c + 1, 1 - P).start()
            else:
                pos_copy(c + 1, 1 - P).start()

            for b in range(BATCH):
                in_copy(c, b, P).wait()
            for b in range(BATCH):
                if P == 0:
                    @pl.when(cc > 0)
                    def _():
                        out_copy(c - 1, b, 1 - P).wait()

                    in_copy(c + 1, b, 1 - P).start()
                else:
                    out_copy(c - 1, b, 1 - P).wait()

                    @pl.when(cc < NUM_CHUNKS // 2 - 1)
                    def _():
                        in_copy(c + 1, b, 1 - P).start()

            def row_body(r, rc):
                for j in range(SLICES):
                    sl = pl.ds(j * LANES, LANES)
                    p = pob[P][r, sl]
                    for b in range(BATCH):
                        iob[b * 2 + P][r, sl] = iob[b * 2 + P][r, sl] + p
                return rc

            lax.fori_loop(0, CHUNK, row_body, 0)

            for b in range(BATCH):
                out_copy(c, b, P).start()
        return carry

    lax.fori_loop(0, NUM_CHUNKS // 2, pair_body, 0)

    for b in range(BATCH):
        out_copy(NUM_CHUNKS - 1, b, 1).wait()


@functools.partial(
    pl.kernel,
    mesh=plsc.VectorSubcoreMesh(core_axis_name="c", subcore_axis_name="s"),
    out_type=jax.ShapeDtypeStruct((BATCH, SENT, FEAT), jnp.float32),
    scratch_types=[
        pltpu.VMEM((10, CHUNK, FEAT), jnp.float32),
        pltpu.SemaphoreType.DMA((10,)),
    ],
)
def _pe(*refs):
    _pe_body(*refs)


def kernel(inputs, pos_weight):
    return _pe(inputs, pos_weight)
